# baseline (speedup 1.0000x reference)
import functools

import jax
import jax.numpy as jnp
from jax import lax
from jax.experimental import pallas as pl
from jax.experimental.pallas import tpu as pltpu
from jax.experimental.pallas import tpu_sc as plsc

BATCH = 16384
HIST = 50
DIM = 32
NUM_EMB = 1000000

_info = plsc.get_sparse_core_info()
NC, NS = _info.num_cores, _info.num_subcores
NW = NC * NS
BAGS_PER_W = BATCH // NW
CHUNK_BAGS = 32
N_CHUNKS = BAGS_PER_W // CHUNK_BAGS


def _ebag_kernel(x_hbm, table_hbm, out_hbm, idx_v, rows_v, out_v, sem):
    wid = lax.axis_index("s") * NC + lax.axis_index("c")

    def chunk_body(c, carry):
        row_base = wid * BAGS_PER_W + c * CHUNK_BAGS

        pltpu.sync_copy(x_hbm.at[pl.ds(row_base, CHUNK_BAGS)], idx_v)

        copies = []
        for r in range(CHUNK_BAGS):
            copies.append(
                pltpu.async_copy(table_hbm.at[idx_v.at[r]], rows_v.at[r], sem)
            )
        for cp in copies:
            cp.wait()

        def bag_body(r, carry2):
            a = [jnp.zeros((16,), jnp.float32) for _ in range(8)]
            for j in range(HIST):
                p = (j % 4) * 2
                a[p] = a[p] + rows_v[r, j, pl.ds(0, 16)]
                a[p + 1] = a[p + 1] + rows_v[r, j, pl.ds(16, 16)]
            s0 = (a[0] + a[2]) + (a[4] + a[6])
            s1 = (a[1] + a[3]) + (a[5] + a[7])
            scale = jnp.float32(1.0 / HIST)
            out_v[r, pl.ds(0, 16)] = s0 * scale
            out_v[r, pl.ds(16, 16)] = s1 * scale
            return carry2

        lax.fori_loop(0, CHUNK_BAGS, bag_body, 0, unroll=False)

        pltpu.sync_copy(out_v, out_hbm.at[pl.ds(row_base, CHUNK_BAGS)])
        return carry

    lax.fori_loop(0, N_CHUNKS, chunk_body, 0, unroll=False)


def _transpose_body(in_ref, out_ref):
    t = in_ref[...].T
    iota_r = lax.broadcasted_iota(jnp.int32, (128, 512), 0)
    iota_c = lax.broadcasted_iota(jnp.int32, (128, 512), 1)
    parts = []
    for s in range(4):
        sel = (iota_c == 4 * iota_r + s).astype(jnp.float32)
        parts.append(
            jax.lax.dot(sel, t, preferred_element_type=jnp.float32)
        )
    out_ref[...] = jnp.concatenate(parts, axis=1)


def _table_to_row_major(table):
    table_t = jnp.swapaxes(table, 0, 1)
    n_steps = (NUM_EMB + 511) // 512
    tq = pl.pallas_call(
        _transpose_body,
        grid=(n_steps,),
        in_specs=[pl.BlockSpec((32, 512), lambda i: (0, i))],
        out_specs=pl.BlockSpec((128, 128), lambda i: (i, 0)),
        out_shape=jax.ShapeDtypeStruct((NUM_EMB // 4, 128), jnp.float32),
    )(table_t)
    return tq.reshape(NUM_EMB, DIM)


@jax.jit
def kernel(x, table):
    table_rm = _table_to_row_major(table)
    mesh = plsc.VectorSubcoreMesh(core_axis_name="c", subcore_axis_name="s")
    run = functools.partial(
        pl.kernel,
        mesh=mesh,
        out_type=jax.ShapeDtypeStruct((BATCH, DIM), jnp.float32),
        scratch_types=[
            pltpu.VMEM((CHUNK_BAGS, HIST), jnp.int32),
            pltpu.VMEM((CHUNK_BAGS, HIST, DIM), jnp.float32),
            pltpu.VMEM((CHUNK_BAGS, DIM), jnp.float32),
            pltpu.SemaphoreType.DMA,
        ],
        compiler_params=pltpu.CompilerParams(use_tc_tiling_on_sc=False),
    )(_ebag_kernel)
    return run(x, table_rm)


# speedup vs baseline: 2.2638x; 2.2638x over previous
import functools

import jax
import jax.numpy as jnp
from jax import lax
from jax.experimental import pallas as pl
from jax.experimental.pallas import tpu as pltpu
from jax.experimental.pallas import tpu_sc as plsc

BATCH = 16384
HIST = 50
DIM = 32
NUM_EMB = 1000000

_info = plsc.get_sparse_core_info()
NC, NS = _info.num_cores, _info.num_subcores
NW = NC * NS
BAGS_PER_W = BATCH // NW
CHUNK_BAGS = 32
N_CHUNKS = BAGS_PER_W // CHUNK_BAGS


def _ebag_kernel(x_hbm, table_hbm, out_hbm, idx_v, rows_v, out_v, sem):
    wid = lax.axis_index("s") * NC + lax.axis_index("c")

    def chunk_body(c, carry):
        row_base = wid * BAGS_PER_W + c * CHUNK_BAGS

        pltpu.sync_copy(x_hbm.at[pl.ds(row_base, CHUNK_BAGS)], idx_v)

        copies = []
        for r in range(CHUNK_BAGS):
            copies.append(
                pltpu.async_copy(table_hbm.at[idx_v.at[r]], rows_v.at[r], sem)
            )
        for cp in copies:
            cp.wait()

        def bag_body(r, carry2):
            a = [jnp.zeros((16,), jnp.float32) for _ in range(8)]
            for j in range(HIST):
                p = (j % 4) * 2
                a[p] = a[p] + rows_v[r, j, pl.ds(0, 16)]
                a[p + 1] = a[p + 1] + rows_v[r, j, pl.ds(16, 16)]
            s0 = (a[0] + a[2]) + (a[4] + a[6])
            s1 = (a[1] + a[3]) + (a[5] + a[7])
            scale = jnp.float32(1.0 / HIST)
            out_v[r, pl.ds(0, 16)] = s0 * scale
            out_v[r, pl.ds(16, 16)] = s1 * scale
            return carry2

        lax.fori_loop(0, CHUNK_BAGS, bag_body, 0, unroll=False)

        pltpu.sync_copy(out_v, out_hbm.at[pl.ds(row_base, CHUNK_BAGS)])
        return carry

    lax.fori_loop(0, N_CHUNKS, chunk_body, 0, unroll=False)


def _transpose_body(in_ref, out_ref):
    t = in_ref[...].T
    iota_r = lax.broadcasted_iota(jnp.int32, (128, 512), 0)
    iota_c = lax.broadcasted_iota(jnp.int32, (128, 512), 1)
    parts = []
    for s in range(4):
        sel = (iota_c == 4 * iota_r + s).astype(jnp.float32)
        parts.append(
            jax.lax.dot(sel, t, preferred_element_type=jnp.float32)
        )
    out_ref[...] = jnp.concatenate(parts, axis=1)


def _table_to_row_major(table):
    return table.reshape(NUM_EMB // 4, 4 * DIM).reshape(NUM_EMB, DIM)


@jax.jit
def kernel(x, table):
    table_rm = _table_to_row_major(table)
    mesh = plsc.VectorSubcoreMesh(core_axis_name="c", subcore_axis_name="s")
    run = functools.partial(
        pl.kernel,
        mesh=mesh,
        out_type=jax.ShapeDtypeStruct((BATCH, DIM), jnp.float32),
        scratch_types=[
            pltpu.VMEM((CHUNK_BAGS, HIST), jnp.int32),
            pltpu.VMEM((CHUNK_BAGS, HIST, DIM), jnp.float32),
            pltpu.VMEM((CHUNK_BAGS, DIM), jnp.float32),
            pltpu.SemaphoreType.DMA,
        ],
        compiler_params=pltpu.CompilerParams(use_tc_tiling_on_sc=False),
    )(_ebag_kernel)
    return run(x, table_rm)
